# Initial kernel scaffold; baseline (speedup 1.0000x reference)
import jax
import jax.numpy as jnp
from jax.experimental import pallas as pl


def kernel(word_batch, tois, word_emb, conv_w, conv_b, fc1_w, fc1_b, fc2_w, fc2_b):
    raise NotImplementedError("write your pallas kernel here")


# speedup vs baseline: 13.2054x; 13.2054x over previous
import jax
import jax.numpy as jnp
from jax import lax
from jax.experimental import pallas as pl
from jax.experimental.pallas import tpu as pltpu

_B, _S, _E = 32, 512, 300
_F, _K = 512, 3
_T = 512
_LABELS = 8
_FF = 3 * _F


def _toi_kernel(x_ref, wk_ref, cb_ref, st_ref, en_ref, w1_ref, b1_ref,
                w2_ref, b2_ref, out_ref):
    x = x_ref[0]
    zrow = jnp.zeros((1, _E), jnp.float32)
    xm1 = jnp.concatenate([zrow, x[:-1, :]], axis=0)
    xp1 = jnp.concatenate([x[1:, :], zrow], axis=0)
    feat = jnp.dot(xm1, wk_ref[0], preferred_element_type=jnp.float32)
    feat = feat + jnp.dot(x, wk_ref[1], preferred_element_type=jnp.float32)
    feat = feat + jnp.dot(xp1, wk_ref[2], preferred_element_type=jnp.float32)
    feat = jnp.maximum(feat + cb_ref[...], 0.0)

    st = st_ref[0]
    en = en_ref[0]
    s_iota = lax.broadcasted_iota(jnp.int32, (_S, _T), 0)
    in_span = (s_iota >= st) & (s_iota < en)
    inv_len = 1.0 / (en - st).astype(jnp.float32)
    m_avg = jnp.where(in_span, jnp.broadcast_to(inv_len, (_S, _T)), 0.0)
    m_s = jnp.where(s_iota == st, 1.0, 0.0)
    m_e = jnp.where(s_iota == en - 1, 1.0, 0.0)

    dn = (((0,), (0,)), ((), ()))
    ps = lax.dot_general(m_s, feat, dn, preferred_element_type=jnp.float32)
    pa = lax.dot_general(m_avg, feat, dn, preferred_element_type=jnp.float32)
    pe = lax.dot_general(m_e, feat, dn, preferred_element_type=jnp.float32)

    h = jnp.dot(ps, w1_ref[0:_F, :], preferred_element_type=jnp.float32)
    h = h + jnp.dot(pa, w1_ref[_F:2 * _F, :], preferred_element_type=jnp.float32)
    h = h + jnp.dot(pe, w1_ref[2 * _F:3 * _F, :], preferred_element_type=jnp.float32)
    h = jnp.maximum(h + b1_ref[...], 0.0)
    out_ref[0] = (jnp.dot(h, w2_ref[...], preferred_element_type=jnp.float32)
                  + b2_ref[...])


def _run(xe, wk, cb, st, en, w1, b1, w2, b2, *, interpret=False):
    return pl.pallas_call(
        _toi_kernel,
        out_shape=jax.ShapeDtypeStruct((_B, _T, _LABELS), jnp.float32),
        grid=(_B,),
        in_specs=[
            pl.BlockSpec((1, _S, _E), lambda b: (b, 0, 0)),
            pl.BlockSpec((_K, _E, _F), lambda b: (0, 0, 0)),
            pl.BlockSpec((1, _F), lambda b: (0, 0)),
            pl.BlockSpec((1, 1, _T), lambda b: (b, 0, 0)),
            pl.BlockSpec((1, 1, _T), lambda b: (b, 0, 0)),
            pl.BlockSpec((_FF, _FF), lambda b: (0, 0)),
            pl.BlockSpec((1, _FF), lambda b: (0, 0)),
            pl.BlockSpec((_FF, _LABELS), lambda b: (0, 0)),
            pl.BlockSpec((1, _LABELS), lambda b: (0, 0)),
        ],
        out_specs=pl.BlockSpec((1, _T, _LABELS), lambda b: (b, 0, 0)),
        compiler_params=pltpu.CompilerParams(
            dimension_semantics=("parallel",),
            vmem_limit_bytes=48 * 1024 * 1024,
        ),
        name="toicnn_fused",
        interpret=interpret,
    )(xe, wk, cb, st, en, w1, b1, w2, b2)


def kernel(word_batch, tois, word_emb, conv_w, conv_b, fc1_w, fc1_b,
           fc2_w, fc2_b):
    xe = word_emb[word_batch.astype(jnp.int32)]
    wk = jnp.transpose(conv_w[:, 0, :, :], (1, 2, 0))
    st = tois[..., 0].astype(jnp.int32).reshape(_B, 1, _T)
    en = tois[..., 1].astype(jnp.int32).reshape(_B, 1, _T)
    out = _run(xe, wk, conv_b.reshape(1, _F), st, en,
               fc1_w.T, fc1_b.reshape(1, _FF),
               fc2_w.T, fc2_b.reshape(1, _LABELS))
    logits = out.reshape(_B * _T, _LABELS)
    toi_section = jnp.cumsum(jnp.full((_B,), _T, dtype=jnp.int32))
    return logits, toi_section


# speedup vs baseline: 19.7226x; 1.4935x over previous
import jax
import jax.numpy as jnp
from jax import lax
from jax.experimental import pallas as pl
from jax.experimental.pallas import tpu as pltpu

_B, _S, _E = 32, 512, 300
_F, _K = 512, 3
_T = 512
_LABELS = 8
_FF = 3 * _F


def _issue_gather(wb_smem, emb_hbm, emb_buf, sems, base, slot):
    for s in range(_S):
        idx = wb_smem[base + s]
        pltpu.make_async_copy(
            emb_hbm.at[idx], emb_buf.at[slot, s], sems.at[slot]).start()


def _wait_gather(emb_buf, sems, slot):
    for s in range(_S):
        pltpu.make_async_copy(
            emb_buf.at[slot, s], emb_buf.at[slot, s], sems.at[slot]).wait()


def _toi_kernel(wb_smem, emb_hbm, wk_ref, cb_ref, st_ref, en_ref, w1_ref,
                b1_ref, w2_ref, b2_ref, out_ref, emb_buf, sems):
    b = pl.program_id(0)
    slot = lax.rem(b, 2)

    @pl.when(b == 0)
    def _():
        _issue_gather(wb_smem, emb_hbm, emb_buf, sems, 0, 0)

    @pl.when(b < _B - 1)
    def _():
        _issue_gather(wb_smem, emb_hbm, emb_buf, sems, (b + 1) * _S, 1 - slot)

    _wait_gather(emb_buf, sems, slot)

    x = emb_buf[slot]
    zrow = jnp.zeros((1, _E), jnp.float32)
    xm1 = jnp.concatenate([zrow, x[:-1, :]], axis=0)
    xp1 = jnp.concatenate([x[1:, :], zrow], axis=0)
    feat = jnp.dot(xm1, wk_ref[0], preferred_element_type=jnp.float32)
    feat = feat + jnp.dot(x, wk_ref[1], preferred_element_type=jnp.float32)
    feat = feat + jnp.dot(xp1, wk_ref[2], preferred_element_type=jnp.float32)
    feat = jnp.maximum(feat + cb_ref[...], 0.0)

    st = st_ref[0]
    en = en_ref[0]
    s_iota = lax.broadcasted_iota(jnp.int32, (_S, _T), 0)
    in_span = (s_iota >= st) & (s_iota < en)
    inv_len = 1.0 / (en - st).astype(jnp.float32)
    m_avg = jnp.where(in_span, jnp.broadcast_to(inv_len, (_S, _T)), 0.0)
    m_s = jnp.where(s_iota == st, 1.0, 0.0)
    m_e = jnp.where(s_iota == en - 1, 1.0, 0.0)

    dn = (((0,), (0,)), ((), ()))
    ps = lax.dot_general(m_s, feat, dn, preferred_element_type=jnp.float32)
    pa = lax.dot_general(m_avg, feat, dn, preferred_element_type=jnp.float32)
    pe = lax.dot_general(m_e, feat, dn, preferred_element_type=jnp.float32)

    h = jnp.dot(ps, w1_ref[0:_F, :], preferred_element_type=jnp.float32)
    h = h + jnp.dot(pa, w1_ref[_F:2 * _F, :], preferred_element_type=jnp.float32)
    h = h + jnp.dot(pe, w1_ref[2 * _F:3 * _F, :], preferred_element_type=jnp.float32)
    h = jnp.maximum(h + b1_ref[...], 0.0)
    out_ref[0] = (jnp.dot(h, w2_ref[...], preferred_element_type=jnp.float32)
                  + b2_ref[...])


def _run(wb, emb, wk, cb, st, en, w1, b1, w2, b2, *, interpret=False):
    grid_spec = pltpu.PrefetchScalarGridSpec(
        num_scalar_prefetch=1,
        grid=(_B,),
        in_specs=[
            pl.BlockSpec(memory_space=pl.ANY),
            pl.BlockSpec((_K, _E, _F), lambda b, wbr: (0, 0, 0)),
            pl.BlockSpec((1, _F), lambda b, wbr: (0, 0)),
            pl.BlockSpec((1, 1, _T), lambda b, wbr: (b, 0, 0)),
            pl.BlockSpec((1, 1, _T), lambda b, wbr: (b, 0, 0)),
            pl.BlockSpec((_FF, _FF), lambda b, wbr: (0, 0)),
            pl.BlockSpec((1, _FF), lambda b, wbr: (0, 0)),
            pl.BlockSpec((_FF, _LABELS), lambda b, wbr: (0, 0)),
            pl.BlockSpec((1, _LABELS), lambda b, wbr: (0, 0)),
        ],
        out_specs=pl.BlockSpec((1, _T, _LABELS), lambda b, wbr: (b, 0, 0)),
        scratch_shapes=[
            pltpu.VMEM((2, _S, _E), jnp.float32),
            pltpu.SemaphoreType.DMA((2,)),
        ],
    )
    return pl.pallas_call(
        _toi_kernel,
        out_shape=jax.ShapeDtypeStruct((_B, _T, _LABELS), jnp.float32),
        grid_spec=grid_spec,
        compiler_params=pltpu.CompilerParams(
            dimension_semantics=("arbitrary",),
            vmem_limit_bytes=48 * 1024 * 1024,
        ),
        name="toicnn_fused",
        interpret=interpret,
    )(wb, emb, wk, cb, st, en, w1, b1, w2, b2)


def kernel(word_batch, tois, word_emb, conv_w, conv_b, fc1_w, fc1_b,
           fc2_w, fc2_b):
    wb = word_batch.astype(jnp.int32).reshape(_B * _S)
    wk = jnp.transpose(conv_w[:, 0, :, :], (1, 2, 0))
    st = tois[..., 0].astype(jnp.int32).reshape(_B, 1, _T)
    en = tois[..., 1].astype(jnp.int32).reshape(_B, 1, _T)
    out = _run(wb, word_emb, wk, conv_b.reshape(1, _F), st, en,
               fc1_w.T, fc1_b.reshape(1, _FF),
               fc2_w.T, fc2_b.reshape(1, _LABELS))
    logits = out.reshape(_B * _T, _LABELS)
    toi_section = jnp.cumsum(jnp.full((_B,), _T, dtype=jnp.int32))
    return logits, toi_section
